# Initial kernel scaffold; baseline (speedup 1.0000x reference)
import jax
import jax.numpy as jnp
from jax.experimental import pallas as pl


def kernel(x_batch, W):
    raise NotImplementedError("write your pallas kernel here")


# speedup vs baseline: 2.1501x; 2.1501x over previous
import functools

import jax
import jax.numpy as jnp
from jax.experimental import pallas as pl
from jax.experimental.pallas import tpu as pltpu

N_NODES = 50000
DIM = 128
BATCH = 4
K = max(1, int(N_NODES * 0.1))
ROW_BLK = 12500
NBLK = N_NODES // ROW_BLK


def _scores_kernel(x_ref, w_ref, s_ref):
    prod = x_ref[0] * w_ref[...]
    s_ref[0] = jnp.sum(prod, axis=1, keepdims=True)


def _pool_kernel(s_full_ref, s_blk_ref, x_ref, pooled_ref, loss_ref, sm):
    b = pl.program_id(0)
    j = pl.program_id(1)
    nf = float(N_NODES)
    kf = float(K)
    eps = 1e-8

    @pl.when(j == 0)
    def _stats():
        S = s_full_ref[0]
        mean = jnp.sum(S) / nf
        var = jnp.sum((S - mean) * (S - mean)) / nf
        inv_std = 1.0 / (jnp.sqrt(var) + eps)

        lo0 = jnp.min(S)
        hi0 = jnp.max(S) + 1.0

        def vbody(_, c):
            lo, hi = c
            mid = 0.5 * (lo + hi)
            cnt = jnp.sum((S >= mid).astype(jnp.float32))
            ge = cnt >= kf
            return (jnp.where(ge, mid, lo), jnp.where(ge, hi, mid))

        vlo, _ = jax.lax.fori_loop(0, 44, vbody, (lo0, hi0))
        v = vlo

        c_gt = jnp.sum((S > v).astype(jnp.float32))
        need = kf - c_gt

        gix = jax.lax.broadcasted_iota(
            jnp.int32, (1, N_NODES), 1).astype(jnp.float32)
        iseq = (S == v)

        def ibody(_, c):
            lo, hi = c
            mid = jnp.floor(0.5 * (lo + hi))
            cnt = jnp.sum(jnp.where(iseq & (gix <= mid), 1.0, 0.0))
            ge = cnt >= need
            return (jnp.where(ge, lo, mid), jnp.where(ge, mid, hi))

        _, tid = jax.lax.fori_loop(0, 17, ibody, (-1.0, nf - 1.0))

        sm[0] = mean
        sm[1] = inv_std
        sm[2] = v
        sm[3] = tid

        sig = jax.nn.sigmoid((S - mean) * inv_std)
        m = (S > v) | (iseq & (gix <= tid))
        contrib = jnp.where(m, jnp.log(sig + eps), jnp.log(1.0 - sig + eps))
        loss_b = -jnp.sum(contrib) / nf

        @pl.when(b == 0)
        def _():
            loss_ref[...] = jnp.zeros((1, 1), jnp.float32)
        loss_ref[...] += jnp.full((1, 1), 1.0 / BATCH) * loss_b

    @pl.when(j == 0)
    def _zero():
        pooled_ref[...] = jnp.zeros_like(pooled_ref)

    mean = sm[0]
    inv_std = sm[1]
    v = sm[2]
    tid = sm[3]

    sb = s_blk_ref[0, 0]
    gixb = (j * ROW_BLK).astype(jnp.float32) + jax.lax.broadcasted_iota(
        jnp.int32, (ROW_BLK, 1), 0).astype(jnp.float32)
    sig = jax.nn.sigmoid((sb - mean) * inv_std)
    m = (sb > v) | ((sb == v) & (gixb <= tid))
    coef = jnp.where(m, sig, 0.0) * (1.0 / kf)
    X = x_ref[0, 0]
    pooled_ref[0] += jnp.sum(X * coef, axis=0, keepdims=True)


@jax.jit
def kernel(x_batch, W):
    x = x_batch.reshape(BATCH, N_NODES, DIM)
    x_blk = x.reshape(BATCH, NBLK, ROW_BLK, DIM)
    w_row = W.reshape(1, DIM)

    scores_col = pl.pallas_call(
        _scores_kernel,
        grid=(BATCH * NBLK,),
        in_specs=[
            pl.BlockSpec((1, ROW_BLK, DIM), lambda i: (i, 0, 0)),
            pl.BlockSpec((1, DIM), lambda i: (0, 0)),
        ],
        out_specs=pl.BlockSpec((1, ROW_BLK, 1), lambda i: (i, 0, 0)),
        out_shape=jax.ShapeDtypeStruct((BATCH * NBLK, ROW_BLK, 1),
                                       jnp.float32),
    )(x.reshape(BATCH * NBLK, ROW_BLK, DIM), w_row)

    s_full = scores_col.reshape(BATCH, 1, N_NODES)
    s_blk = scores_col.reshape(BATCH, NBLK, ROW_BLK, 1)

    pooled, loss = pl.pallas_call(
        _pool_kernel,
        grid=(BATCH, NBLK),
        in_specs=[
            pl.BlockSpec((1, 1, N_NODES), lambda b, j: (b, 0, 0)),
            pl.BlockSpec((1, 1, ROW_BLK, 1), lambda b, j: (b, j, 0, 0)),
            pl.BlockSpec((1, 1, ROW_BLK, DIM), lambda b, j: (b, j, 0, 0)),
        ],
        out_specs=[
            pl.BlockSpec((1, 1, DIM), lambda b, j: (b, 0, 0)),
            pl.BlockSpec((1, 1), lambda b, j: (0, 0)),
        ],
        out_shape=[
            jax.ShapeDtypeStruct((BATCH, 1, DIM), jnp.float32),
            jax.ShapeDtypeStruct((1, 1), jnp.float32),
        ],
        scratch_shapes=[pltpu.SMEM((8,), jnp.float32)],
    )(s_full, s_blk, x_blk)

    return pooled.reshape(BATCH, DIM), loss[0, 0]


# speedup vs baseline: 2.3546x; 1.0951x over previous
import functools

import jax
import jax.numpy as jnp
from jax.experimental import pallas as pl
from jax.experimental.pallas import tpu as pltpu

N_NODES = 50000
DIM = 128
BATCH = 4
K = max(1, int(N_NODES * 0.1))
ROW_BLK = 12500
NBLK = N_NODES // ROW_BLK


def _scores_kernel(x_ref, w_ref, s_ref):
    prod = x_ref[0] * w_ref[...]
    s = jnp.sum(prod, axis=1)
    s_ref[0] = s.reshape(1, ROW_BLK)


def _pool_kernel(s_full_ref, s_blk_ref, x_ref, pooled_ref, loss_ref, sm):
    b = pl.program_id(0)
    j = pl.program_id(1)
    nf = float(N_NODES)
    kf = float(K)
    eps = 1e-8

    @pl.when(j == 0)
    def _stats():
        S = s_full_ref[0]
        mean = jnp.sum(S) / nf
        var = jnp.sum((S - mean) * (S - mean)) / nf
        inv_std = 1.0 / (jnp.sqrt(var) + eps)

        lo0 = jnp.min(S)
        hi0 = jnp.max(S) + 1.0

        def vbody(_, c):
            lo, hi = c
            mid = 0.5 * (lo + hi)
            cnt = jnp.sum((S >= mid).astype(jnp.float32))
            ge = cnt >= kf
            return (jnp.where(ge, mid, lo), jnp.where(ge, hi, mid))

        vlo, _ = jax.lax.fori_loop(0, 44, vbody, (lo0, hi0))
        v = vlo

        c_gt = jnp.sum((S > v).astype(jnp.float32))
        need = kf - c_gt

        gix = jax.lax.broadcasted_iota(
            jnp.int32, (1, N_NODES), 1).astype(jnp.float32)
        iseq = (S == v)

        def ibody(_, c):
            lo, hi = c
            mid = jnp.floor(0.5 * (lo + hi))
            cnt = jnp.sum(jnp.where(iseq & (gix <= mid), 1.0, 0.0))
            ge = cnt >= need
            return (jnp.where(ge, lo, mid), jnp.where(ge, mid, hi))

        _, tid = jax.lax.fori_loop(0, 17, ibody, (-1.0, nf - 1.0))

        sm[0] = mean
        sm[1] = inv_std
        sm[2] = v
        sm[3] = tid

        sig = jax.nn.sigmoid((S - mean) * inv_std)
        m = (S > v) | (iseq & (gix <= tid))
        contrib = jnp.where(m, jnp.log(sig + eps), jnp.log(1.0 - sig + eps))
        loss_b = -jnp.sum(contrib) / nf

        @pl.when(b == 0)
        def _():
            loss_ref[...] = jnp.zeros((1, 1), jnp.float32)
        loss_ref[...] += jnp.full((1, 1), 1.0 / BATCH) * loss_b

    @pl.when(j == 0)
    def _zero():
        pooled_ref[...] = jnp.zeros_like(pooled_ref)

    mean = sm[0]
    inv_std = sm[1]
    v = sm[2]
    tid = sm[3]

    sb = s_blk_ref[0, 0]
    gixb = (j * ROW_BLK).astype(jnp.float32) + jax.lax.broadcasted_iota(
        jnp.int32, (1, ROW_BLK), 1).astype(jnp.float32)
    sig = jax.nn.sigmoid((sb - mean) * inv_std)
    m = (sb > v) | ((sb == v) & (gixb <= tid))
    coef_l = jnp.where(m, sig, 0.0) * (1.0 / kf)
    coef = coef_l.reshape(ROW_BLK, 1)
    X = x_ref[0, 0]
    pooled_ref[0] += jnp.sum(X * coef, axis=0, keepdims=True)


@jax.jit
def kernel(x_batch, W):
    x = x_batch.reshape(BATCH, N_NODES, DIM)
    x_blk = x.reshape(BATCH, NBLK, ROW_BLK, DIM)
    w_row = W.reshape(1, DIM)

    scores_col = pl.pallas_call(
        _scores_kernel,
        grid=(BATCH * NBLK,),
        in_specs=[
            pl.BlockSpec((1, ROW_BLK, DIM), lambda i: (i, 0, 0)),
            pl.BlockSpec((1, DIM), lambda i: (0, 0)),
        ],
        out_specs=pl.BlockSpec((1, 1, ROW_BLK), lambda i: (i, 0, 0)),
        out_shape=jax.ShapeDtypeStruct((BATCH * NBLK, 1, ROW_BLK),
                                       jnp.float32),
    )(x.reshape(BATCH * NBLK, ROW_BLK, DIM), w_row)

    s_full = scores_col.reshape(BATCH, 1, N_NODES)
    s_blk = scores_col.reshape(BATCH, NBLK, 1, ROW_BLK)

    pooled, loss = pl.pallas_call(
        _pool_kernel,
        grid=(BATCH, NBLK),
        in_specs=[
            pl.BlockSpec((1, 1, N_NODES), lambda b, j: (b, 0, 0)),
            pl.BlockSpec((1, 1, 1, ROW_BLK), lambda b, j: (b, j, 0, 0)),
            pl.BlockSpec((1, 1, ROW_BLK, DIM), lambda b, j: (b, j, 0, 0)),
        ],
        out_specs=[
            pl.BlockSpec((1, 1, DIM), lambda b, j: (b, 0, 0)),
            pl.BlockSpec((1, 1), lambda b, j: (0, 0)),
        ],
        out_shape=[
            jax.ShapeDtypeStruct((BATCH, 1, DIM), jnp.float32),
            jax.ShapeDtypeStruct((1, 1), jnp.float32),
        ],
        scratch_shapes=[pltpu.SMEM((8,), jnp.float32)],
    )(s_full, s_blk, x_blk)

    return pooled.reshape(BATCH, DIM), loss[0, 0]


# speedup vs baseline: 3.7145x; 1.5776x over previous
import functools

import jax
import jax.numpy as jnp
from jax.experimental import pallas as pl
from jax.experimental.pallas import tpu as pltpu

N_NODES = 50000
DIM = 128
BATCH = 4
K = max(1, int(N_NODES * 0.1))
ROW_BLK = 10000
NBLK = N_NODES // ROW_BLK


def _scores_kernel(x_ref, w_ref, s_ref):
    prod = x_ref[...] * w_ref[...]
    s = jnp.sum(prod, axis=1)
    s_ref[0] = s.reshape(1, ROW_BLK)


def _pool_kernel(s_full_ref, s_blk_ref, x_ref, pooled_ref, loss_ref, sm):
    b = pl.program_id(0)
    j = pl.program_id(1)
    nf = float(N_NODES)
    kf = float(K)
    eps = 1e-8

    @pl.when(j == 0)
    def _stats():
        S = s_full_ref[0]
        mean = jnp.sum(S) / nf
        var = jnp.sum((S - mean) * (S - mean)) / nf
        inv_std = 1.0 / (jnp.sqrt(var) + eps)

        lo0 = jnp.min(S)
        hi0 = jnp.max(S) + 1.0

        def vbody(_, c):
            lo, hi = c
            mid = 0.5 * (lo + hi)
            cnt = jnp.sum((S >= mid).astype(jnp.float32))
            ge = cnt >= kf
            return (jnp.where(ge, mid, lo), jnp.where(ge, hi, mid))

        vlo, _ = jax.lax.fori_loop(0, 44, vbody, (lo0, hi0))
        v = vlo

        c_gt = jnp.sum((S > v).astype(jnp.float32))
        need = kf - c_gt

        gix = jax.lax.broadcasted_iota(
            jnp.int32, (1, N_NODES), 1).astype(jnp.float32)
        iseq = (S == v)

        def ibody(_, c):
            lo, hi = c
            mid = jnp.floor(0.5 * (lo + hi))
            cnt = jnp.sum(jnp.where(iseq & (gix <= mid), 1.0, 0.0))
            ge = cnt >= need
            return (jnp.where(ge, lo, mid), jnp.where(ge, mid, hi))

        _, tid = jax.lax.fori_loop(0, 17, ibody, (-1.0, nf - 1.0))

        sm[0] = mean
        sm[1] = inv_std
        sm[2] = v
        sm[3] = tid

        sig = jax.nn.sigmoid((S - mean) * inv_std)
        m = (S > v) | (iseq & (gix <= tid))
        contrib = jnp.where(m, jnp.log(sig + eps), jnp.log(1.0 - sig + eps))
        loss_b = -jnp.sum(contrib) / nf

        @pl.when(b == 0)
        def _():
            loss_ref[...] = jnp.zeros((1, 1), jnp.float32)
        loss_ref[...] += jnp.full((1, 1), 1.0 / BATCH) * loss_b

    @pl.when(j == 0)
    def _zero():
        pooled_ref[...] = jnp.zeros_like(pooled_ref)

    mean = sm[0]
    inv_std = sm[1]
    v = sm[2]
    tid = sm[3]

    sb = s_blk_ref[0, 0]
    gixb = (j * ROW_BLK).astype(jnp.float32) + jax.lax.broadcasted_iota(
        jnp.int32, (1, ROW_BLK), 1).astype(jnp.float32)
    sig = jax.nn.sigmoid((sb - mean) * inv_std)
    m = (sb > v) | ((sb == v) & (gixb <= tid))
    coef_l = jnp.where(m, sig, 0.0) * (1.0 / kf)
    coef = coef_l.reshape(ROW_BLK, 1)
    X = x_ref[...]
    pooled_ref[0] += jnp.sum(X * coef, axis=0, keepdims=True)


@jax.jit
def kernel(x_batch, W):
    w_row = W.reshape(1, DIM)

    scores_l = pl.pallas_call(
        _scores_kernel,
        grid=(BATCH * NBLK,),
        in_specs=[
            pl.BlockSpec((ROW_BLK, DIM), lambda i: (i, 0)),
            pl.BlockSpec((1, DIM), lambda i: (0, 0)),
        ],
        out_specs=pl.BlockSpec((1, 1, ROW_BLK), lambda i: (i, 0, 0)),
        out_shape=jax.ShapeDtypeStruct((BATCH * NBLK, 1, ROW_BLK),
                                       jnp.float32),
    )(x_batch, w_row)

    s_full = scores_l.reshape(BATCH, 1, N_NODES)
    s_blk = scores_l.reshape(BATCH, NBLK, 1, ROW_BLK)

    pooled, loss = pl.pallas_call(
        _pool_kernel,
        grid=(BATCH, NBLK),
        in_specs=[
            pl.BlockSpec((1, 1, N_NODES), lambda b, j: (b, 0, 0)),
            pl.BlockSpec((1, 1, 1, ROW_BLK), lambda b, j: (b, j, 0, 0)),
            pl.BlockSpec((ROW_BLK, DIM), lambda b, j: (b * NBLK + j, 0)),
        ],
        out_specs=[
            pl.BlockSpec((1, 1, DIM), lambda b, j: (b, 0, 0)),
            pl.BlockSpec((1, 1), lambda b, j: (0, 0)),
        ],
        out_shape=[
            jax.ShapeDtypeStruct((BATCH, 1, DIM), jnp.float32),
            jax.ShapeDtypeStruct((1, 1), jnp.float32),
        ],
        scratch_shapes=[pltpu.SMEM((8,), jnp.float32)],
    )(s_full, s_blk, x_batch)

    return pooled.reshape(BATCH, DIM), loss[0, 0]
